# baseline (speedup 1.0000x reference)
import functools

import jax
import jax.numpy as jnp
from jax import lax
from jax.experimental import pallas as pl
from jax.experimental.pallas import tpu as pltpu
from jax.experimental.pallas import tpu_sc as plsc

NC = 2
NS = 16
NW = NC * NS
CHW = 1024
HCAP = 2064


def _emb_body(ids_hbm, tab_t_hbm, outa_hbm, outb_hbm, rows_hbm,
              idsv, chva, chvb, hid, hb, cid, cb, rows_v, cols_v, zero_v,
              sema, semb, sem2, *tmps, B, V, D):
  sid = lax.axis_index("s")
  c = lax.axis_index("c")
  wid = sid * NC + c
  ncols_pad = ((V + 127) // 128) * 128
  nch = (V + CHW - 1) // CHW
  cpt = (nch + NW - 1) // NW
  smax = ncols_pad - CHW
  rows16 = lax.iota(jnp.int32, 16)

  lo_w = wid * cpt * CHW
  hi_w = jnp.minimum(lo_w + cpt * CHW, V)

  pltpu.sync_copy(ids_hbm, idsv)

  def pf(i, p):
    v = idsv[pl.ds(i * 16, 16)]
    v = jnp.where(v == -1, V - 1, v)
    m = (v >= lo_w) & (v < hi_w)
    plsc.store_compressed(hid.at[pl.ds(p, 16)], v, mask=m)
    plsc.store_compressed(hb.at[pl.ds(p, 16)], rows16 + i * 16, mask=m)
    return p + plsc.all_reduce_population_count(m)[0]

  nh = lax.fori_loop(0, B // 16, pf, 0)

  for i in range(256):
    zero_v[pl.ds(i * 16, 16)] = jnp.zeros((16,), jnp.float32)
  half = c * (B * D)
  slice_w = B * D // NS

  def zq(q, carry):
    pltpu.sync_copy(
        zero_v, rows_hbm.at[pl.ds(half + sid * slice_w + q * 4096, 4096)])
    return carry

  lax.fori_loop(0, slice_w // 4096, zq, 0)
  plsc.subcore_barrier()

  def start_of(t):
    k_c = jnp.minimum(wid * cpt + t, nch - 1)
    return k_c, pl.multiple_of(jnp.minimum(k_c * CHW, smax), 128)

  def fire(t, dst, sem):
    _, s_k = start_of(t)
    pltpu.async_copy(tab_t_hbm.at[:, pl.ds(s_k, CHW)], dst, sem)

  def drain(dst, sem):
    pltpu.make_async_copy(tab_t_hbm.at[:, pl.ds(0, CHW)], dst, sem).wait()

  def process(t, buf):
    k_c, s_k = start_of(t)
    lo_k = k_c * CHW
    hi_k = jnp.minimum(lo_k + CHW, V)

    def cscan(i, cptr):
      v = hid[pl.ds(i * 16, 16)]
      b = hb[pl.ds(i * 16, 16)]
      live = (rows16 + i * 16) < nh
      m = live & (v >= lo_k) & (v < hi_k)
      plsc.store_compressed(cid.at[pl.ds(cptr, 16)], v, mask=m)
      plsc.store_compressed(cb.at[pl.ds(cptr, 16)], b, mask=m)
      return cptr + plsc.all_reduce_population_count(m)[0]

    cn = lax.fori_loop(0, (nh + 15) // 16, cscan, 0)

    def proc(g, inner):
      idv = cid[pl.ds(g * 16, 16)]
      bv = cb[pl.ds(g * 16, 16)]
      m = (rows16 + g * 16) < cn
      idv = jnp.where(m, idv, jnp.full((16,), idv[0], jnp.int32))
      bv = jnp.where(m, bv, jnp.full((16,), bv[0], jnp.int32))
      lv = idv - s_k
      copies = []
      for j in range(16):
        lane = jnp.full((16,), lv[j], jnp.int32)
        lo16 = plsc.load_gather(buf, [rows16, lane])
        hi16 = plsc.load_gather(buf, [rows16 + 16, lane])
        tmps[j][pl.ds(0, 16)] = lo16
        tmps[j][pl.ds(16, 16)] = hi16
        copies.append(
            pltpu.async_copy(tmps[j], rows_hbm.at[pl.ds(half + bv[j] * D, D)], sem2))
      for cpy in copies:
        cpy.wait()
      return inner

    lax.fori_loop(0, (cn + 15) // 16, proc, 0)

  fire(0, chva, sema)

  def pair(p, carry):
    fire(2 * p + 1, chvb, semb)
    drain(chva, sema)
    process(2 * p, chva)
    fire(2 * p + 2, chva, sema)
    drain(chvb, semb)
    process(2 * p + 1, chvb)
    return carry

  lax.fori_loop(0, (cpt + 1) // 2, pair, 0)
  drain(chva, sema)

  plsc.subcore_barrier()

  for blk in range(8):
    row0 = (8 * sid + blk) * 128
    pltpu.sync_copy(rows_hbm.at[pl.ds(half + row0 * D, 128 * D)], rows_v)

    def tr(r, carry):
      rr = jnp.full((16,), r, jnp.int32)
      lo16 = rows_v[pl.ds(r * D, 16)]
      hi16 = rows_v[pl.ds(r * D + 16, 16)]
      plsc.store_scatter(cols_v, [rows16, rr], lo16)
      plsc.store_scatter(cols_v, [rows16 + 16, rr], hi16)
      return carry

    lax.fori_loop(0, 128, tr, 0)
    col0 = pl.multiple_of(row0, 128)

    @pl.when(c == 0)
    def _():
      pltpu.sync_copy(cols_v, outa_hbm.at[:, pl.ds(col0, 128)])

    @pl.when(c == 1)
    def _():
      pltpu.sync_copy(cols_v, outb_hbm.at[:, pl.ds(col0, 128)])


@jax.jit
def kernel(class_ids, table):
  B = class_ids.shape[0]
  V, D = table.shape

  ids = class_ids.astype(jnp.int32)
  tab_t = table.T

  body = functools.partial(_emb_body, B=B, V=V, D=D)
  k = pl.kernel(
      body,
      out_type=(jax.ShapeDtypeStruct((D, B), jnp.float32),
                jax.ShapeDtypeStruct((D, B), jnp.float32),
                jax.ShapeDtypeStruct((2 * B * D,), jnp.float32)),
      mesh=plsc.VectorSubcoreMesh(core_axis_name="c", subcore_axis_name="s"),
      compiler_params=pltpu.CompilerParams(needs_layout_passes=False),
      scratch_types=[
          pltpu.VMEM((B,), jnp.int32),
          pltpu.VMEM((D, CHW), jnp.float32),
          pltpu.VMEM((D, CHW), jnp.float32),
          pltpu.VMEM((HCAP,), jnp.int32),
          pltpu.VMEM((HCAP,), jnp.int32),
          pltpu.VMEM((HCAP,), jnp.int32),
          pltpu.VMEM((HCAP,), jnp.int32),
          pltpu.VMEM((128 * 32,), jnp.float32),
          pltpu.VMEM((D, 128), jnp.float32),
          pltpu.VMEM((4096,), jnp.float32),
          pltpu.SemaphoreType.DMA,
          pltpu.SemaphoreType.DMA,
          pltpu.SemaphoreType.DMA,
      ] + [pltpu.VMEM((D,), jnp.float32) for _ in range(16)],
  )
  outa, outb, _ = k(ids, tab_t)
  return (outa + outb).T


# speedup vs baseline: 1.2318x; 1.2318x over previous
import functools

import jax
import jax.numpy as jnp
from jax import lax
from jax.experimental import pallas as pl
from jax.experimental.pallas import tpu as pltpu
from jax.experimental.pallas import tpu_sc as plsc

NC = 2
NS = 16
NW = NC * NS
CHW = 1024
HCAP = 2064


def _emb_body(ids_hbm, tab_t_hbm, out_hbm,
              idsv, chva, chvb, hid, hb, cid, cb,
              sema, semb, sem2, *tmps, B, V, D):
  sid = lax.axis_index("s")
  c = lax.axis_index("c")
  wid = sid * NC + c
  ncols_pad = ((V + 127) // 128) * 128
  nch = (V + CHW - 1) // CHW
  cpt = (nch + NW - 1) // NW
  smax = ncols_pad - CHW
  rows16 = lax.iota(jnp.int32, 16)

  lo_w = wid * cpt * CHW
  hi_w = jnp.minimum(lo_w + cpt * CHW, V)

  pltpu.sync_copy(ids_hbm, idsv)

  def pf(i, p):
    v = idsv[pl.ds(i * 16, 16)]
    v = jnp.where(v == -1, V - 1, v)
    m = (v >= lo_w) & (v < hi_w)
    plsc.store_compressed(hid.at[pl.ds(p, 16)], v, mask=m)
    plsc.store_compressed(hb.at[pl.ds(p, 16)], rows16 + i * 16, mask=m)
    return p + plsc.all_reduce_population_count(m)[0]

  nh = lax.fori_loop(0, B // 16, pf, 0)

  def start_of(t):
    k_c = jnp.minimum(wid * cpt + t, nch - 1)
    return k_c, pl.multiple_of(jnp.minimum(k_c * CHW, smax), 128)

  def fire(t, dst, sem):
    _, s_k = start_of(t)
    pltpu.async_copy(tab_t_hbm.at[:, pl.ds(s_k, CHW)], dst, sem)

  def drain(dst, sem):
    pltpu.make_async_copy(tab_t_hbm.at[:, pl.ds(0, CHW)], dst, sem).wait()

  def process(t, buf):
    k_c, s_k = start_of(t)
    lo_k = k_c * CHW
    hi_k = jnp.minimum(lo_k + CHW, V)

    def cscan(i, cptr):
      v = hid[pl.ds(i * 16, 16)]
      b = hb[pl.ds(i * 16, 16)]
      live = (rows16 + i * 16) < nh
      m = live & (v >= lo_k) & (v < hi_k)
      plsc.store_compressed(cid.at[pl.ds(cptr, 16)], v, mask=m)
      plsc.store_compressed(cb.at[pl.ds(cptr, 16)], b, mask=m)
      return cptr + plsc.all_reduce_population_count(m)[0]

    cn = lax.fori_loop(0, (nh + 15) // 16, cscan, 0)

    def proc(g, inner):
      idv = cid[pl.ds(g * 16, 16)]
      bv = cb[pl.ds(g * 16, 16)]
      m = (rows16 + g * 16) < cn
      idv = jnp.where(m, idv, jnp.full((16,), idv[0], jnp.int32))
      bv = jnp.where(m, bv, jnp.full((16,), bv[0], jnp.int32))
      lv = idv - s_k
      copies = []
      for j in range(16):
        lane = jnp.full((16,), lv[j], jnp.int32)
        lo16 = plsc.load_gather(buf, [rows16, lane])
        hi16 = plsc.load_gather(buf, [rows16 + 16, lane])
        tmps[j][pl.ds(0, 16)] = lo16
        tmps[j][pl.ds(16, 16)] = hi16
        copies.append(
            pltpu.async_copy(tmps[j], out_hbm.at[pl.ds(bv[j] * D, D)], sem2))
      for cpy in copies:
        cpy.wait()
      return inner

    lax.fori_loop(0, (cn + 15) // 16, proc, 0)

  fire(0, chva, sema)

  def pair(p, carry):
    fire(2 * p + 1, chvb, semb)
    drain(chva, sema)
    process(2 * p, chva)
    fire(2 * p + 2, chva, sema)
    drain(chvb, semb)
    process(2 * p + 1, chvb)
    return carry

  lax.fori_loop(0, (cpt + 1) // 2, pair, 0)
  drain(chva, sema)


@jax.jit
def kernel(class_ids, table):
  B = class_ids.shape[0]
  V, D = table.shape

  ids = class_ids.astype(jnp.int32)
  tab_t = table.T

  body = functools.partial(_emb_body, B=B, V=V, D=D)
  k = pl.kernel(
      body,
      out_type=jax.ShapeDtypeStruct((B * D,), jnp.float32),
      mesh=plsc.VectorSubcoreMesh(core_axis_name="c", subcore_axis_name="s"),
      compiler_params=pltpu.CompilerParams(needs_layout_passes=False),
      scratch_types=[
          pltpu.VMEM((B,), jnp.int32),
          pltpu.VMEM((D, CHW), jnp.float32),
          pltpu.VMEM((D, CHW), jnp.float32),
          pltpu.VMEM((HCAP,), jnp.int32),
          pltpu.VMEM((HCAP,), jnp.int32),
          pltpu.VMEM((HCAP,), jnp.int32),
          pltpu.VMEM((HCAP,), jnp.int32),
          pltpu.SemaphoreType.DMA,
          pltpu.SemaphoreType.DMA,
          pltpu.SemaphoreType.DMA,
      ] + [pltpu.VMEM((D,), jnp.float32) for _ in range(16)],
  )
  out1d = k(ids, tab_t)
  return out1d.reshape(B, D)
